# baseline (speedup 1.0000x reference)
import functools

import jax
import jax.numpy as jnp
from jax import lax
from jax.experimental import pallas as pl
from jax.experimental.pallas import tpu as pltpu
from jax.experimental.pallas import tpu_sc as plsc

TOTAL = 32768
B = 16
D = 256
L = 16
NC = 2
NS = 16
NW = NC * NS
SC_ROWS = 20480
TC_ROWS = TOTAL - SC_ROWS
TC_BLOCK = 1024
ROWS_PER_TILE = SC_ROWS // NW
NBUF = 4
CHUNK_ROWS = 64
N_CHUNKS = ROWS_PER_TILE // CHUNK_ROWS
SUB_ROWS = 32
SUBS_PER_CHUNK = CHUNK_ROWS // SUB_ROWS


def _sc_body(feat_hbm, ids_hbm, out_hbm, buf_v, ids_v, acc_v,
             sem0, sem1, sem2, sem3):
    wid = lax.axis_index("c") * NS + lax.axis_index("s")
    base_row = wid * ROWS_PER_TILE

    def start_chunk(c, slot_static):
        sem = (sem0, sem1, sem2, sem3)[slot_static]
        src = feat_hbm.at[pl.ds(base_row + c * CHUNK_ROWS, CHUNK_ROWS)]
        dst = buf_v.at[pl.ds(slot_static * CHUNK_ROWS, CHUNK_ROWS)]
        pltpu.async_copy(src, dst, sem)

    def wait_chunk(slot_static):
        sem = (sem0, sem1, sem2, sem3)[slot_static]
        src = feat_hbm.at[pl.ds(0, CHUNK_ROWS)]
        dst = buf_v.at[pl.ds(slot_static * CHUNK_ROWS, CHUNK_ROWS)]
        pltpu.make_async_copy(src, dst, sem).wait()

    def select4(slot, f0, f1, f2, f3):
        def lo(u):
            return lax.cond(slot == 0, f0, f1, u)

        def hi(u):
            return lax.cond(slot == 2, f2, f3, u)
        lax.cond(slot < 2, lo, hi, 0)

    for s in range(NBUF):
        start_chunk(s, s)
    pltpu.sync_copy(ids_hbm.at[pl.ds(base_row, ROWS_PER_TILE)], ids_v)

    def zbody(i, carry):
        zv = jnp.zeros((L,), jnp.float32)
        for j in range(D // L):
            acc_v[i, pl.ds(j * L, L)] = zv
        return carry
    lax.fori_loop(0, B, zbody, 0)

    def sub_block(c, slot, h):
        irow = c * CHUNK_ROWS + h * SUB_ROWS
        brow = slot * CHUNK_ROWS + h * SUB_ROWS
        iva = ids_v[pl.ds(irow, 16)]
        ivb = ids_v[pl.ds(irow + SUB_ROWS - 16, 16)]
        s_a = iva[0]
        s_b = ivb[15]

        def fast(carry):
            for j in range(D // L):
                m = buf_v[brow, pl.ds(j * L, L)]
                for r in range(1, SUB_ROWS):
                    m = jnp.maximum(m, buf_v[brow + r, pl.ds(j * L, L)])
                cur = acc_v[s_a, pl.ds(j * L, L)]
                acc_v[s_a, pl.ds(j * L, L)] = jnp.maximum(cur, m)
            return carry

        def slow(carry):
            for gg in range(SUB_ROWS // 16):
                ivg = ids_v[pl.ds(irow + gg * 16, 16)]
                for r in range(16):
                    s_r = ivg[r]
                    row = brow + gg * 16 + r
                    for j in range(D // L):
                        cur = acc_v[s_r, pl.ds(j * L, L)]
                        acc_v[s_r, pl.ds(j * L, L)] = jnp.maximum(
                            cur, buf_v[row, pl.ds(j * L, L)])
            return carry

        lax.cond(s_a == s_b, fast, slow, 0)

    def chunk_body(c, carry):
        slot = c & (NBUF - 1)

        def w(i):
            def f(u):
                wait_chunk(i)
                return u
            return f
        select4(slot, w(0), w(1), w(2), w(3))

        def sbody(h, hc):
            sub_block(c, slot, h)
            return hc
        lax.fori_loop(0, SUBS_PER_CHUNK, sbody, 0)

        @pl.when(c < N_CHUNKS - NBUF)
        def _start_next():
            def g(i):
                def f(u):
                    start_chunk(c + NBUF, i)
                    return u
                return f
            select4(slot, g(0), g(1), g(2), g(3))
        return carry
    lax.fori_loop(0, N_CHUNKS, chunk_body, 0)

    pltpu.sync_copy(acc_v, out_hbm.at[wid])


_sc_segmax = functools.partial(
    pl.kernel,
    out_type=jax.ShapeDtypeStruct((NW, B, D), jnp.float32),
    mesh=plsc.VectorSubcoreMesh(
        core_axis_name="c", subcore_axis_name="s",
        num_cores=NC, num_subcores=NS),
    scratch_types=[
        pltpu.VMEM((NBUF * CHUNK_ROWS, D), jnp.float32),
        pltpu.VMEM((ROWS_PER_TILE,), jnp.int32),
        pltpu.VMEM((B, D), jnp.float32),
        pltpu.SemaphoreType.DMA,
        pltpu.SemaphoreType.DMA,
        pltpu.SemaphoreType.DMA,
        pltpu.SemaphoreType.DMA,
    ],
    compiler_params=pltpu.CompilerParams(use_tc_tiling_on_sc=True),
)(_sc_body)


def _tc_segmax_body(feat_ref, ids_ref, out_ref):
    i = pl.program_id(0)

    @pl.when(i == 0)
    def _init():
        out_ref[...] = jnp.zeros((B, D), jnp.float32)

    f = feat_ref[...]
    ids_col = ids_ref[...]
    smin = jnp.min(ids_col)
    smax = jnp.max(ids_col)
    for s in range(B):
        @pl.when((smin <= s) & (s <= smax))
        def _seg():
            contrib = jnp.max(jnp.where(ids_col == s, f, 0.0), axis=0)
            out_ref[s:s + 1, :] = jnp.maximum(out_ref[s:s + 1, :],
                                              contrib[None, :])


def _tc_segmax(feat_tail, ids2d_tail):
    return pl.pallas_call(
        _tc_segmax_body,
        grid=(TC_ROWS // TC_BLOCK,),
        in_specs=[
            pl.BlockSpec((TC_BLOCK, D), lambda i: (i, 0)),
            pl.BlockSpec((TC_BLOCK, 1), lambda i: (i, 0)),
        ],
        out_specs=pl.BlockSpec((B, D), lambda i: (0, 0)),
        out_shape=jax.ShapeDtypeStruct((B, D), jnp.float32),
    )(feat_tail, ids2d_tail)


def _tc_body(part_ref, tcpart_ref, w1_ref, g_ref, bt_ref, w2_ref, b2_ref,
             pooled_ref, proj_ref):
    part = part_ref[...]
    pooled = jnp.maximum(jnp.max(part, axis=0), tcpart_ref[...])
    pooled_ref[...] = pooled
    h = lax.dot_general(pooled, w1_ref[...],
                        (((1,), (1,)), ((), ())),
                        preferred_element_type=jnp.float32)
    mean = jnp.mean(h, axis=0, keepdims=True)
    var = jnp.mean((h - mean) ** 2, axis=0, keepdims=True)
    hn = (h - mean) / jnp.sqrt(var + 1e-5) * g_ref[...] + bt_ref[...]
    hr = jnp.maximum(hn, 0.0)
    proj_ref[...] = lax.dot_general(hr, w2_ref[...],
                                    (((1,), (1,)), ((), ())),
                                    preferred_element_type=jnp.float32) \
        + b2_ref[...]


def _tc_mlp(part3, tcpart, W1, gamma, beta, W2, b2):
    return pl.pallas_call(
        _tc_body,
        out_shape=[
            jax.ShapeDtypeStruct((B, D), jnp.float32),
            jax.ShapeDtypeStruct((B, 128), jnp.float32),
        ],
    )(part3, tcpart, W1, gamma.reshape(1, D), beta.reshape(1, D),
      W2, b2.reshape(1, 128))


def kernel(features, segment_ids, W1, gamma, beta, W2, b2):
    ids32 = segment_ids.astype(jnp.int32)
    partials = _sc_segmax(features, ids32)
    feat_tail = lax.slice(features, (SC_ROWS, 0), (TOTAL, D))
    ids_tail = lax.slice(ids32, (SC_ROWS,), (TOTAL,)).reshape(TC_ROWS, 1)
    tcpart = _tc_segmax(feat_tail, ids_tail)
    pooled, proj = _tc_mlp(partials, tcpart, W1, gamma, beta, W2, b2)
    return (pooled, proj)


# speedup vs baseline: 1.0789x; 1.0789x over previous
import functools

import jax
import jax.numpy as jnp
from jax import lax
from jax.experimental import pallas as pl
from jax.experimental.pallas import tpu as pltpu
from jax.experimental.pallas import tpu_sc as plsc

TOTAL = 32768
B = 16
D = 256
L = 16
NC = 2
NS = 16
NW = NC * NS
ROWS_PER_TILE = TOTAL // NW
NBUF = 4
CHUNK_ROWS = 64
N_CHUNKS = ROWS_PER_TILE // CHUNK_ROWS
SUB_ROWS = 32
SUBS_PER_CHUNK = CHUNK_ROWS // SUB_ROWS


def _sc_body(feat_hbm, ids_hbm, out_hbm, buf_v, ids_v, acc_v,
             sem0, sem1, sem2, sem3):
    wid = lax.axis_index("c") * NS + lax.axis_index("s")
    base_row = wid * ROWS_PER_TILE

    def start_chunk(c, slot_static):
        sem = (sem0, sem1, sem2, sem3)[slot_static]
        src = feat_hbm.at[pl.ds(base_row + c * CHUNK_ROWS, CHUNK_ROWS)]
        dst = buf_v.at[pl.ds(slot_static * CHUNK_ROWS, CHUNK_ROWS)]
        pltpu.async_copy(src, dst, sem)

    def wait_chunk(slot_static):
        sem = (sem0, sem1, sem2, sem3)[slot_static]
        src = feat_hbm.at[pl.ds(0, CHUNK_ROWS)]
        dst = buf_v.at[pl.ds(slot_static * CHUNK_ROWS, CHUNK_ROWS)]
        pltpu.make_async_copy(src, dst, sem).wait()

    def select4(slot, f0, f1, f2, f3):
        def lo(u):
            return lax.cond(slot == 0, f0, f1, u)

        def hi(u):
            return lax.cond(slot == 2, f2, f3, u)
        lax.cond(slot < 2, lo, hi, 0)

    for s in range(NBUF):
        start_chunk(s, s)
    pltpu.sync_copy(ids_hbm.at[pl.ds(base_row, ROWS_PER_TILE)], ids_v)

    def zbody(i, carry):
        zv = jnp.zeros((L,), jnp.float32)
        for j in range(D // L):
            acc_v[i, pl.ds(j * L, L)] = zv
        return carry
    lax.fori_loop(0, B, zbody, 0)

    def sub_block(c, slot, h):
        irow = c * CHUNK_ROWS + h * SUB_ROWS
        brow = slot * CHUNK_ROWS + h * SUB_ROWS
        iva = ids_v[pl.ds(irow, 16)]
        ivb = ids_v[pl.ds(irow + SUB_ROWS - 16, 16)]
        s_a = iva[0]
        s_b = ivb[15]

        def fast(carry):
            for j in range(D // L):
                m = buf_v[brow, pl.ds(j * L, L)]
                for r in range(1, SUB_ROWS):
                    m = jnp.maximum(m, buf_v[brow + r, pl.ds(j * L, L)])
                cur = acc_v[s_a, pl.ds(j * L, L)]
                acc_v[s_a, pl.ds(j * L, L)] = jnp.maximum(cur, m)
            return carry

        def slow(carry):
            for gg in range(SUB_ROWS // 16):
                ivg = ids_v[pl.ds(irow + gg * 16, 16)]
                for r in range(16):
                    s_r = ivg[r]
                    row = brow + gg * 16 + r
                    for j in range(D // L):
                        cur = acc_v[s_r, pl.ds(j * L, L)]
                        acc_v[s_r, pl.ds(j * L, L)] = jnp.maximum(
                            cur, buf_v[row, pl.ds(j * L, L)])
            return carry

        lax.cond(s_a == s_b, fast, slow, 0)

    def chunk_body(c, carry):
        slot = c & (NBUF - 1)

        def w(i):
            def f(u):
                wait_chunk(i)
                return u
            return f
        select4(slot, w(0), w(1), w(2), w(3))

        def sbody(h, hc):
            sub_block(c, slot, h)
            return hc
        lax.fori_loop(0, SUBS_PER_CHUNK, sbody, 0)

        @pl.when(c < N_CHUNKS - NBUF)
        def _start_next():
            def g(i):
                def f(u):
                    start_chunk(c + NBUF, i)
                    return u
                return f
            select4(slot, g(0), g(1), g(2), g(3))
        return carry
    lax.fori_loop(0, N_CHUNKS, chunk_body, 0)

    pltpu.sync_copy(acc_v, out_hbm.at[wid])


_sc_segmax = functools.partial(
    pl.kernel,
    out_type=jax.ShapeDtypeStruct((NW, B, D), jnp.float32),
    mesh=plsc.VectorSubcoreMesh(
        core_axis_name="c", subcore_axis_name="s",
        num_cores=NC, num_subcores=NS),
    scratch_types=[
        pltpu.VMEM((NBUF * CHUNK_ROWS, D), jnp.float32),
        pltpu.VMEM((ROWS_PER_TILE,), jnp.int32),
        pltpu.VMEM((B, D), jnp.float32),
        pltpu.SemaphoreType.DMA,
        pltpu.SemaphoreType.DMA,
        pltpu.SemaphoreType.DMA,
        pltpu.SemaphoreType.DMA,
    ],
    compiler_params=pltpu.CompilerParams(use_tc_tiling_on_sc=True),
)(_sc_body)


def _tc_body(part_ref, w1_ref, g_ref, bt_ref, w2_ref, b2_ref,
             pooled_ref, proj_ref):
    part = part_ref[...]
    pooled = jnp.max(part, axis=0)
    pooled_ref[...] = pooled
    h = lax.dot_general(pooled, w1_ref[...],
                        (((1,), (1,)), ((), ())),
                        preferred_element_type=jnp.float32)
    mean = jnp.mean(h, axis=0, keepdims=True)
    var = jnp.mean((h - mean) ** 2, axis=0, keepdims=True)
    hn = (h - mean) / jnp.sqrt(var + 1e-5) * g_ref[...] + bt_ref[...]
    hr = jnp.maximum(hn, 0.0)
    proj_ref[...] = lax.dot_general(hr, w2_ref[...],
                                    (((1,), (1,)), ((), ())),
                                    preferred_element_type=jnp.float32) \
        + b2_ref[...]


def _tc_mlp(part3, W1, gamma, beta, W2, b2):
    return pl.pallas_call(
        _tc_body,
        out_shape=[
            jax.ShapeDtypeStruct((B, D), jnp.float32),
            jax.ShapeDtypeStruct((B, 128), jnp.float32),
        ],
    )(part3, W1, gamma.reshape(1, D), beta.reshape(1, D),
      W2, b2.reshape(1, 128))


def kernel(features, segment_ids, W1, gamma, beta, W2, b2):
    ids32 = segment_ids.astype(jnp.int32)
    partials = _sc_segmax(features, ids32)
    pooled, proj = _tc_mlp(partials, W1, gamma, beta, W2, b2)
    return (pooled, proj)
